# baseline (speedup 1.0000x reference)
import functools

import jax
import jax.numpy as jnp
from jax import lax
from jax.experimental import pallas as pl
from jax.experimental.pallas import tpu as pltpu
from jax.experimental.pallas import tpu_sc as plsc

N = 10000
D = 128
NPACK = D // 2
WPT = 4
EDGE_CHUNK = 16000
UNROLL = 4


def _dotT(a, b):
    return lax.dot_general(a, b, (((1,), (1,)), ((), ())),
                           preferred_element_type=jnp.float32)


def _dot0T(a, b):
    return lax.dot_general(a, b, (((0,), (1,)), ((), ())),
                           preferred_element_type=jnp.float32)


def _pack_bf16(hpT):
    u_lo = lax.bitcast_convert_type(hpT[:NPACK], jnp.uint32)
    u_hi = lax.bitcast_convert_type(hpT[NPACK:], jnp.uint32)

    def rne(u):
        return (u + jnp.uint32(0x7FFF) + ((u >> 16) & jnp.uint32(1))) \
            & jnp.uint32(0xFFFF0000)

    w = rne(u_hi) | (rne(u_lo) >> 16)
    return lax.bitcast_convert_type(w, jnp.int32)


def _unpack_bf16(w2):
    wu = lax.bitcast_convert_type(w2, jnp.uint32)
    lo = lax.bitcast_convert_type(wu << 16, jnp.float32)
    hi = lax.bitcast_convert_type(wu & jnp.uint32(0xFFFF0000), jnp.float32)
    return jnp.concatenate([jnp.maximum(lo[0], lo[1]),
                            jnp.maximum(hi[0], hi[1])], axis=0)


def _pool_projT_body(h_ref, Wp_ref, bpT_ref, ei_ref, hpP_ref, sd_ref,
                     flg_ref):
    hpT = jnp.maximum(_dotT(Wp_ref[...], h_ref[...]) + bpT_ref[...], 0.0)
    hpP_ref[...] = _pack_bf16(hpT)
    ei = ei_ref[...]
    sd_ref[...] = ei[0] * jnp.int32(16384) + ei[1]
    E = ei.shape[1]
    d2 = ei[1].reshape(E // 128, 128)
    lane = lax.broadcasted_iota(jnp.int32, (E // 128, 128), 1)
    dup = jnp.zeros(d2.shape, jnp.bool_)
    for k in range(1, 16):
        shifted = jnp.pad(d2, ((0, 0), (k, 0)))[:, :128]
        dup = jnp.logical_or(
            dup, jnp.logical_and(d2 == shifted, (lane % 16) >= k))
    grp = lane[0] // 16
    grpmat = (grp[:, None] == lax.broadcasted_iota(jnp.int32, (128, 8), 1)
              ).astype(jnp.float32)
    cnt = lax.dot_general(dup.astype(jnp.float32), grpmat,
                          (((1,), (0,)), ((), ())),
                          preferred_element_type=jnp.float32)
    flg_ref[...] = (cnt > 0.0).astype(jnp.int32)


def _mid_body(x_ref, poolP_ref, Ws_ref, Wn_ref, b_ref, Wp1_ref, bpT1_ref,
              h1_ref, hp1P_ref):
    poolT = _unpack_bf16(poolP_ref[...])
    out0 = _dotT(x_ref[...], Ws_ref[...]) + _dot0T(poolT, Wn_ref[...])
    out0 = out0 + b_ref[...]
    h = jnp.maximum(out0, 0.0)
    nrm = jnp.sqrt(jnp.sum(h * h, axis=1, keepdims=True))
    h1 = h / jnp.maximum(nrm, 1e-12)
    h1_ref[...] = h1
    hp1P_ref[...] = _pack_bf16(
        jnp.maximum(_dotT(Wp1_ref[...], h1) + bpT1_ref[...], 0.0))


def _final_body(h_ref, poolP_ref, Ws_ref, Wn_ref, b_ref, out_ref):
    poolT = _unpack_bf16(poolP_ref[...])
    out = _dotT(h_ref[...], Ws_ref[...]) + _dot0T(poolT, Wn_ref[...])
    out_ref[...] = out + b_ref[...]


def _tc_call(body, out_shapes, *args):
    return pl.pallas_call(body, out_shape=out_shapes)(*args)


def _lo_f32(w):
    return plsc.bitcast(w << 16, jnp.float32)


def _hi_f32(w):
    return plsc.bitcast(w & jnp.int32(-65536), jnp.float32)


def _make_seg_max(E):
    EH = E // 2
    assert EH % (2 * EDGE_CHUNK) == 0 and EDGE_CHUNK % (16 * UNROLL) == 0
    n_pairs = EH // (2 * EDGE_CHUNK)
    blocks = EDGE_CHUNK // (16 * UNROLL)
    mesh = plsc.VectorSubcoreMesh(core_axis_name="c", subcore_axis_name="s")

    @functools.partial(
        pl.kernel,
        mesh=mesh,
        out_type=jax.ShapeDtypeStruct((2 * NPACK * N,), jnp.int32),
        compiler_params=pltpu.CompilerParams(needs_layout_passes=False),
        scratch_types=(
            [pltpu.VMEM((N,), jnp.int32) for _ in range(WPT)]
            + [pltpu.VMEM((N,), jnp.int32) for _ in range(WPT)]
            + [pltpu.VMEM((EDGE_CHUNK,), jnp.int32),
               pltpu.VMEM((EDGE_CHUNK,), jnp.int32),
               pltpu.VMEM((E // 32 + 32,), jnp.int32),
               pltpu.SemaphoreType.DMA,
               pltpu.SemaphoreType.DMA]
        ),
    )
    def seg_max(hpP_hbm, sd_hbm, flg_hbm, poolP_hbm,
                hp0, hp1, hp2, hp3, pool0, pool1, pool2, pool3,
                sd_a, sd_b, fl_v,
                sem_a, sem_b):
        hp_refs = (hp0, hp1, hp2, hp3)
        pool_refs = (pool0, pool1, pool2, pool3)

        cid = lax.axis_index("c")
        sid = lax.axis_index("s")
        base = sid * (WPT * N)
        edge0 = cid * EH
        out_base = cid * (NPACK * N) + base

        for c in range(WPT):
            pltpu.sync_copy(hpP_hbm.at[pl.ds(base + c * N, N)], hp_refs[c])
        pltpu.sync_copy(flg_hbm.at[pl.ds(cid * (EH // 16), EH // 16)],
                        fl_v.at[pl.ds(0, EH // 16)])

        zero16 = jnp.zeros((16,), jnp.int32)

        def zero_body(i, carry):
            for j in range(5):
                for c in range(WPT):
                    pool_refs[c][pl.ds((i * 5 + j) * 16, 16)] = zero16
            return carry

        lax.fori_loop(0, N // 80, zero_body, 0)

        lanes = lax.iota(jnp.int32, 16)
        no_dup = lanes == lanes
        lane_lt4 = lanes < UNROLL

        def blk_dirty(g0):
            fvec = fl_v[pl.ds(g0, 16)]
            return jnp.any(jnp.logical_and(fvec != 0, lane_lt4))

        def block_core(sd_v, goff, b, dirty):
                e0 = b * (16 * UNROLL)
                sds = [sd_v[pl.ds(e0 + u * 16, 16)] for u in range(UNROLL)]
                svs = [lax.shift_right_logical(sds[u], 14)
                       for u in range(UNROLL)]
                dvs = [sds[u] & jnp.int32(16383) for u in range(UNROLL)]

                @pl.when(jnp.logical_not(dirty))
                def _fast():
                    vals_all = [[plsc.load_gather(hp_refs[c], [svs[u]])
                                 for c in range(WPT)]
                                for u in range(UNROLL)]
                    for u in range(UNROLL):
                        d = dvs[u]
                        vals = vals_all[u]
                        curs = [plsc.load_gather(pool_refs[c], [d])
                                for c in range(WPT)]
                        news = [
                            plsc.bitcast(
                                jnp.maximum(
                                    plsc.bitcast(curs[c], jnp.bfloat16),
                                    plsc.bitcast(vals[c], jnp.bfloat16)),
                                jnp.int32)
                            for c in range(WPT)
                        ]
                        for c in range(WPT):
                            plsc.store_scatter(pool_refs[c], [d], news[c])

                @pl.when(dirty)
                def _repair():
                    for u in range(UNROLL):
                        s = svs[u]
                        d = dvs[u]
                        vals = [plsc.load_gather(hp_refs[c], [s])
                                for c in range(WPT)]
                        for k in range(1, 16):
                            idx = (lanes + k) & jnp.int32(15)
                            dk = jnp.take_along_axis(d, idx, axis=0)
                            same = dk == d
                            for c in range(WPT):
                                vk = jnp.take_along_axis(vals[c], idx,
                                                         axis=0)
                                mx = plsc.bitcast(
                                    jnp.maximum(
                                        plsc.bitcast(vals[c], jnp.bfloat16),
                                        plsc.bitcast(vk, jnp.bfloat16)),
                                    jnp.int32)
                                vals[c] = jnp.where(same, mx, vals[c])
                        curs = [plsc.load_gather(pool_refs[c], [d])
                                for c in range(WPT)]
                        news = [
                            plsc.bitcast(
                                jnp.maximum(
                                    plsc.bitcast(curs[c], jnp.bfloat16),
                                    plsc.bitcast(vals[c], jnp.bfloat16)),
                                jnp.int32)
                            for c in range(WPT)
                        ]
                        for c in range(WPT):
                            plsc.store_scatter(pool_refs[c], [d], news[c])

        GPC = EDGE_CHUNK // 16

        def start(ci, buf, sem):
            pltpu.async_copy(
                sd_hbm.at[pl.ds(edge0 + ci * EDGE_CHUNK, EDGE_CHUNK)],
                buf, sem)

        def wait(buf, sem):
            pltpu.make_async_copy(
                sd_hbm.at[pl.ds(0, EDGE_CHUNK)], buf, sem).wait()

        assert blocks % 2 == 0
        QUAD = 4
        n_quads, rem2 = divmod(blocks, QUAD)
        assert rem2 in (0, 2)

        def quad_body(sd_v, goff, bb, carry):
            nxt = tuple(
                blk_dirty(goff + (QUAD * bb + QUAD + j) * UNROLL)
                for j in range(QUAD))
            for j in range(QUAD):
                block_core(sd_v, goff, QUAD * bb + j, carry[j])
            return nxt

        def tail2(sd_v, goff, carry):
            nA = blk_dirty(goff + (blocks + 0) * UNROLL)
            nB = blk_dirty(goff + (blocks + 1) * UNROLL)
            nC = blk_dirty(goff + (blocks + 2) * UNROLL)
            nD = blk_dirty(goff + (blocks + 3) * UNROLL)
            block_core(sd_v, goff, blocks - 2, carry[0])
            block_core(sd_v, goff, blocks - 1, carry[1])
            return (nA, nB, nC, nD)

        def chunk_run(sd_v, goff, carry):
            carry = lax.fori_loop(
                0, n_quads, lambda b, c: quad_body(sd_v, goff, b, c), carry)
            if rem2:
                carry = tail2(sd_v, goff, carry)
            return carry

        start(0, sd_a, sem_a)

        def pair_body(i, dirty):
            wait(sd_a, sem_a)
            start(2 * i + 1, sd_b, sem_b)
            dirty = chunk_run(sd_a, (2 * i) * GPC, dirty)
            wait(sd_b, sem_b)

            @pl.when(i + 1 < n_pairs)
            def _():
                start(2 * i + 2, sd_a, sem_a)

            return chunk_run(sd_b, (2 * i + 1) * GPC, dirty)

        lax.fori_loop(0, n_pairs, pair_body,
                      tuple(blk_dirty(j * UNROLL) for j in range(QUAD)))

        for c in range(WPT):
            pltpu.sync_copy(pool_refs[c],
                            poolP_hbm.at[pl.ds(out_base + c * N, N)])

    return seg_max


def kernel(inputs, edge_index, Wp0, bp0, Wn0, Ws0, b0, Wp1, bp1, Wn1, Ws1, b1):
    x = inputs
    E = edge_index.shape[1]
    seg_max = _make_seg_max(E)

    bp0T = bp0[:, None]
    bp1T = bp1[:, None]
    b0_r = b0[None, :]
    b1_r = b1[None, :]

    hp0P, sd, flg = _tc_call(_pool_projT_body,
                             (jax.ShapeDtypeStruct((NPACK, N), jnp.int32),
                              jax.ShapeDtypeStruct((E,), jnp.int32),
                              jax.ShapeDtypeStruct((E // 128, 8), jnp.int32)),
                             x, Wp0, bp0T, edge_index)
    flg = flg.reshape(E // 16)
    pool0P = seg_max(hp0P.reshape(NPACK * N), sd, flg).reshape(2, NPACK, N)
    h1, hp1P = _tc_call(
        _mid_body,
        (jax.ShapeDtypeStruct((N, D), jnp.float32),
         jax.ShapeDtypeStruct((NPACK, N), jnp.int32)),
        x, pool0P, Ws0, Wn0, b0_r, Wp1, bp1T)
    pool1P = seg_max(hp1P.reshape(NPACK * N), sd, flg).reshape(2, NPACK, N)
    out = _tc_call(_final_body,
                   jax.ShapeDtypeStruct((N, D), jnp.float32),
                   h1, pool1P, Ws1, Wn1, b1_r)
    return (out, h1)


# speedup vs baseline: 1.0138x; 1.0138x over previous
import functools

import jax
import jax.numpy as jnp
from jax import lax
from jax.experimental import pallas as pl
from jax.experimental.pallas import tpu as pltpu
from jax.experimental.pallas import tpu_sc as plsc

N = 10000
D = 128
NPACK = D // 2
WPT = 4
EDGE_CHUNK = 16000
UNROLL = 4


def _dotT(a, b):
    return lax.dot_general(a, b, (((1,), (1,)), ((), ())),
                           preferred_element_type=jnp.float32)


def _dot0T(a, b):
    return lax.dot_general(a, b, (((0,), (1,)), ((), ())),
                           preferred_element_type=jnp.float32)


def _pack_bf16(hpT):
    u_lo = lax.bitcast_convert_type(hpT[:NPACK], jnp.uint32)
    u_hi = lax.bitcast_convert_type(hpT[NPACK:], jnp.uint32)

    def rne(u):
        return (u + jnp.uint32(0x7FFF) + ((u >> 16) & jnp.uint32(1))) \
            & jnp.uint32(0xFFFF0000)

    w = rne(u_hi) | (rne(u_lo) >> 16)
    return lax.bitcast_convert_type(w, jnp.int32)


def _unpack_bf16(w2):
    wu = lax.bitcast_convert_type(w2, jnp.uint32)
    lo = lax.bitcast_convert_type(wu << 16, jnp.float32)
    hi = lax.bitcast_convert_type(wu & jnp.uint32(0xFFFF0000), jnp.float32)
    return jnp.concatenate([jnp.maximum(lo[0], lo[1]),
                            jnp.maximum(hi[0], hi[1])], axis=0)


def _pool_projT_body(h_ref, Wp_ref, bpT_ref, ei_ref, hpP_ref, sd_ref,
                     flg_ref):
    hpT = jnp.maximum(_dotT(Wp_ref[...], h_ref[...]) + bpT_ref[...], 0.0)
    hpP_ref[...] = _pack_bf16(hpT)
    ei = ei_ref[...]
    sd_ref[...] = ei[0] * jnp.int32(16384) + ei[1]
    E = ei.shape[1]
    d2 = ei[1].reshape(E // 128, 128)
    lane = lax.broadcasted_iota(jnp.int32, (E // 128, 128), 1)
    dup = jnp.zeros(d2.shape, jnp.bool_)
    for k in range(1, 16):
        shifted = jnp.pad(d2, ((0, 0), (k, 0)))[:, :128]
        dup = jnp.logical_or(
            dup, jnp.logical_and(d2 == shifted, (lane % 16) >= k))
    grp = lane[0] // 16
    grpmat = (grp[:, None] == lax.broadcasted_iota(jnp.int32, (128, 8), 1)
              ).astype(jnp.float32)
    cnt = lax.dot_general(dup.astype(jnp.float32), grpmat,
                          (((1,), (0,)), ((), ())),
                          preferred_element_type=jnp.float32)
    flg_ref[...] = (cnt > 0.0).astype(jnp.int32)


def _mid_body(x_ref, poolP_ref, Ws_ref, Wn_ref, b_ref, Wp1_ref, bpT1_ref,
              h1_ref, hp1P_ref):
    poolT = _unpack_bf16(poolP_ref[...])
    out0 = _dotT(x_ref[...], Ws_ref[...]) + _dot0T(poolT, Wn_ref[...])
    out0 = out0 + b_ref[...]
    h = jnp.maximum(out0, 0.0)
    nrm = jnp.sqrt(jnp.sum(h * h, axis=1, keepdims=True))
    h1 = h / jnp.maximum(nrm, 1e-12)
    h1_ref[...] = h1
    hp1P_ref[...] = _pack_bf16(
        jnp.maximum(_dotT(Wp1_ref[...], h1) + bpT1_ref[...], 0.0))


def _final_body(h_ref, poolP_ref, Ws_ref, Wn_ref, b_ref, out_ref):
    poolT = _unpack_bf16(poolP_ref[...])
    out = _dotT(h_ref[...], Ws_ref[...]) + _dot0T(poolT, Wn_ref[...])
    out_ref[...] = out + b_ref[...]


def _tc_call(body, out_shapes, *args):
    return pl.pallas_call(body, out_shape=out_shapes)(*args)


def _lo_f32(w):
    return plsc.bitcast(w << 16, jnp.float32)


def _hi_f32(w):
    return plsc.bitcast(w & jnp.int32(-65536), jnp.float32)


def _make_seg_max(E):
    EH = E // 2
    assert EH % (2 * EDGE_CHUNK) == 0 and EDGE_CHUNK % (16 * UNROLL) == 0
    n_pairs = EH // (2 * EDGE_CHUNK)
    blocks = EDGE_CHUNK // (16 * UNROLL)
    mesh = plsc.VectorSubcoreMesh(core_axis_name="c", subcore_axis_name="s")

    @functools.partial(
        pl.kernel,
        mesh=mesh,
        out_type=jax.ShapeDtypeStruct((2 * NPACK * N,), jnp.int32),
        compiler_params=pltpu.CompilerParams(needs_layout_passes=False),
        scratch_types=(
            [pltpu.VMEM((N,), jnp.int32) for _ in range(WPT)]
            + [pltpu.VMEM((N,), jnp.int32) for _ in range(WPT)]
            + [pltpu.VMEM((EDGE_CHUNK,), jnp.int32),
               pltpu.VMEM((EDGE_CHUNK,), jnp.int32),
               pltpu.VMEM((E // 32 + 32,), jnp.int32),
               pltpu.SemaphoreType.DMA,
               pltpu.SemaphoreType.DMA]
        ),
    )
    def seg_max(hpP_hbm, sd_hbm, flg_hbm, poolP_hbm,
                hp0, hp1, hp2, hp3, pool0, pool1, pool2, pool3,
                sd_a, sd_b, fl_v,
                sem_a, sem_b):
        hp_refs = (hp0, hp1, hp2, hp3)
        pool_refs = (pool0, pool1, pool2, pool3)

        cid = lax.axis_index("c")
        sid = lax.axis_index("s")
        base = sid * (WPT * N)
        edge0 = cid * EH
        out_base = cid * (NPACK * N) + base

        for c in range(WPT):
            pltpu.sync_copy(hpP_hbm.at[pl.ds(base + c * N, N)], hp_refs[c])
        pltpu.sync_copy(flg_hbm.at[pl.ds(cid * (EH // 16), EH // 16)],
                        fl_v.at[pl.ds(0, EH // 16)])

        zero16 = jnp.zeros((16,), jnp.int32)

        def zero_body(i, carry):
            for j in range(5):
                for c in range(WPT):
                    pool_refs[c][pl.ds((i * 5 + j) * 16, 16)] = zero16
            return carry

        lax.fori_loop(0, N // 80, zero_body, 0)

        lanes = lax.iota(jnp.int32, 16)
        no_dup = lanes == lanes
        lane_lt4 = lanes < UNROLL

        def blk_dirty(g0):
            fvec = fl_v[pl.ds(g0, 16)]
            return jnp.any(jnp.logical_and(fvec != 0, lane_lt4))

        def block_core(sd_v, goff, b, dirty):
                e0 = b * (16 * UNROLL)
                sds = [sd_v[pl.ds(e0 + u * 16, 16)] for u in range(UNROLL)]
                svs = [lax.shift_right_logical(sds[u], 14)
                       for u in range(UNROLL)]
                dvs = [sds[u] & jnp.int32(16383) for u in range(UNROLL)]

                @pl.when(jnp.logical_not(dirty))
                def _fast():
                    vals_all = [[plsc.load_gather(hp_refs[c], [svs[u]])
                                 for c in range(WPT)]
                                for u in range(UNROLL)]
                    for u in range(UNROLL):
                        d = dvs[u]
                        vals = vals_all[u]
                        curs = [plsc.load_gather(pool_refs[c], [d])
                                for c in range(WPT)]
                        news = [
                            plsc.bitcast(
                                jnp.maximum(
                                    plsc.bitcast(curs[c], jnp.bfloat16),
                                    plsc.bitcast(vals[c], jnp.bfloat16)),
                                jnp.int32)
                            for c in range(WPT)
                        ]
                        for c in range(WPT):
                            plsc.store_scatter(pool_refs[c], [d], news[c])

                @pl.when(dirty)
                def _repair():
                    for u in range(UNROLL):
                        s = svs[u]
                        d = dvs[u]
                        vals = [plsc.load_gather(hp_refs[c], [s])
                                for c in range(WPT)]
                        for k in range(1, 16):
                            idx = (lanes + k) & jnp.int32(15)
                            dk = jnp.take_along_axis(d, idx, axis=0)
                            same = dk == d
                            for c in range(WPT):
                                vk = jnp.take_along_axis(vals[c], idx,
                                                         axis=0)
                                mx = plsc.bitcast(
                                    jnp.maximum(
                                        plsc.bitcast(vals[c], jnp.bfloat16),
                                        plsc.bitcast(vk, jnp.bfloat16)),
                                    jnp.int32)
                                vals[c] = jnp.where(same, mx, vals[c])
                        curs = [plsc.load_gather(pool_refs[c], [d])
                                for c in range(WPT)]
                        news = [
                            plsc.bitcast(
                                jnp.maximum(
                                    plsc.bitcast(curs[c], jnp.bfloat16),
                                    plsc.bitcast(vals[c], jnp.bfloat16)),
                                jnp.int32)
                            for c in range(WPT)
                        ]
                        for c in range(WPT):
                            plsc.store_scatter(pool_refs[c], [d], news[c])

        GPC = EDGE_CHUNK // 16

        def start(ci, buf, sem):
            pltpu.async_copy(
                sd_hbm.at[pl.ds(edge0 + ci * EDGE_CHUNK, EDGE_CHUNK)],
                buf, sem)

        def wait(buf, sem):
            pltpu.make_async_copy(
                sd_hbm.at[pl.ds(0, EDGE_CHUNK)], buf, sem).wait()

        assert blocks % 2 == 0

        def dbl_body(sd_v, goff, bb, carry):
            dA, dB = carry
            nA = blk_dirty(goff + (2 * bb + 2) * UNROLL)
            nB = blk_dirty(goff + (2 * bb + 3) * UNROLL)
            block_core(sd_v, goff, 2 * bb, dA)
            block_core(sd_v, goff, 2 * bb + 1, dB)
            return (nA, nB)

        start(0, sd_a, sem_a)

        def pair_body(i, dirty):
            wait(sd_a, sem_a)
            start(2 * i + 1, sd_b, sem_b)
            ga = (2 * i) * GPC
            dirty = lax.fori_loop(0, blocks // 2,
                                  lambda b, c: dbl_body(sd_a, ga, b, c),
                                  dirty)
            wait(sd_b, sem_b)

            @pl.when(i + 1 < n_pairs)
            def _():
                start(2 * i + 2, sd_a, sem_a)

            gb = (2 * i + 1) * GPC
            return lax.fori_loop(0, blocks // 2,
                                 lambda b, c: dbl_body(sd_b, gb, b, c),
                                 dirty)

        lax.fori_loop(0, n_pairs, pair_body,
                      (blk_dirty(0), blk_dirty(UNROLL)))

        for c in range(WPT):
            pltpu.sync_copy(pool_refs[c],
                            poolP_hbm.at[pl.ds(out_base + c * N, N)])

    return seg_max


def kernel(inputs, edge_index, Wp0, bp0, Wn0, Ws0, b0, Wp1, bp1, Wn1, Ws1, b1):
    x = inputs
    E = edge_index.shape[1]
    seg_max = _make_seg_max(E)

    bp0T = bp0[:, None]
    bp1T = bp1[:, None]
    b0_r = b0[None, :]
    b1_r = b1[None, :]

    hp0P, sd, flg = _tc_call(_pool_projT_body,
                             (jax.ShapeDtypeStruct((NPACK, N), jnp.int32),
                              jax.ShapeDtypeStruct((E,), jnp.int32),
                              jax.ShapeDtypeStruct((E // 128, 8), jnp.int32)),
                             x, Wp0, bp0T, edge_index)
    flg = flg.reshape(E // 16)
    pool0P = seg_max(hp0P.reshape(NPACK * N), sd, flg).reshape(2, NPACK, N)
    h1, hp1P = _tc_call(
        _mid_body,
        (jax.ShapeDtypeStruct((N, D), jnp.float32),
         jax.ShapeDtypeStruct((NPACK, N), jnp.int32)),
        x, pool0P, Ws0, Wn0, b0_r, Wp1, bp1T)
    pool1P = seg_max(hp1P.reshape(NPACK * N), sd, flg).reshape(2, NPACK, N)
    out = _tc_call(_final_body,
                   jax.ShapeDtypeStruct((N, D), jnp.float32),
                   h1, pool1P, Ws1, Wn1, b1_r)
    return (out, h1)
